# baseline (speedup 1.0000x reference)
import functools

import jax
import jax.numpy as jnp
from jax import lax
from jax.experimental import pallas as pl
from jax.experimental.pallas import tpu as pltpu
from jax.experimental.pallas import tpu_sc as plsc

_LANES = 16
_NUM_CORES = 2
_NUM_SUBCORES = 16
_MAX_IDX = 128


@functools.lru_cache(maxsize=None)
def _build_sc_kernel(N, Q, D, C):
    NW = _NUM_CORES * _NUM_SUBCORES
    TPW = N // NW
    S = TPW // C
    R = C * Q
    G = R // _MAX_IDX

    mesh = plsc.VectorSubcoreMesh(
        core_axis_name="c", subcore_axis_name="s",
        num_cores=_NUM_CORES, num_subcores=_NUM_SUBCORES)

    @functools.partial(
        pl.kernel,
        out_type=jax.ShapeDtypeStruct((N, D // 2), jnp.int32),
        mesh=mesh,
        scratch_types=[
            pltpu.VMEM((TPW * Q,), jnp.int32),
            pltpu.VMEM((R, D // 2), jnp.int32),
            pltpu.VMEM((R, D // 2), jnp.int32),
            pltpu.VMEM((C, D // 2), jnp.int32),
            pltpu.VMEM((C, D // 2), jnp.int32),
            pltpu.SemaphoreType.DMA,
            pltpu.SemaphoreType.DMA,
            pltpu.SemaphoreType.DMA,
            pltpu.SemaphoreType.DMA,
        ],
    )
    def sc_kernel(tab_hbm, fidx_hbm, out_hbm, fidx_v,
                  rows0, rows1, ob0, ob1, g0, g1, o0, o1):
        wid = lax.axis_index("s") * _NUM_CORES + lax.axis_index("c")
        tbase = wid * TPW
        bufs = ((rows0, g0, ob0, o0), (rows1, g1, ob1, o1))

        pltpu.sync_copy(fidx_hbm.at[pl.ds(tbase * Q, TPW * Q)], fidx_v)

        def gather(s, rows, gsem):
            for h in range(G):
                pltpu.async_copy(
                    tab_hbm.at[fidx_v.at[pl.ds(s * R + h * _MAX_IDX, _MAX_IDX)]],
                    rows.at[pl.ds(h * _MAX_IDX, _MAX_IDX)], gsem)

        def gather_wait(s, rows, gsem):
            for h in range(G):
                pltpu.make_async_copy(
                    tab_hbm.at[fidx_v.at[pl.ds(s * R + h * _MAX_IDX, _MAX_IDX)]],
                    rows.at[pl.ds(h * _MAX_IDX, _MAX_IDX)], gsem).wait()

        himask = jnp.full((_LANES,), -65536, jnp.int32)
        half = jnp.full((_LANES,), 0x8000, jnp.int32)

        def compute(rows_v, ob_v):
            @plsc.parallel_loop(0, C, unroll=2)
            def token(t):
                r0 = t * Q
                for jj in range(D // (2 * _LANES)):
                    sl = pl.ds(jj * _LANES, _LANES)
                    vs = [rows_v[r0 + q, sl] for q in range(Q)]
                    los = [lax.bitcast_convert_type(v << 16, jnp.float32)
                           for v in vs]
                    his = [lax.bitcast_convert_type(v & himask, jnp.float32)
                           for v in vs]
                    while len(los) > 1:
                        los = [los[i] + los[i + 1] for i in range(0, len(los), 2)]
                        his = [his[i] + his[i + 1] for i in range(0, len(his), 2)]
                    il = lax.bitcast_convert_type(los[0], jnp.int32)
                    ih = lax.bitcast_convert_type(his[0], jnp.int32)
                    packed = (lax.shift_right_logical(il + half, 16)
                              | ((ih + half) & himask))
                    ob_v[t, sl] = packed

        gather(0, rows0, g0)
        gather(1, rows1, g1)

        def body(i, carry):
            for b in range(2):
                rows, gsem, ob, osem = bufs[b]
                s = 2 * i + b
                gather_wait(s, rows, gsem)
                @pl.when(i > 0)
                def _():
                    pltpu.make_async_copy(
                        ob, out_hbm.at[pl.ds(tbase + (s - 2) * C, C)], osem).wait()
                compute(rows, ob)
                pltpu.async_copy(ob, out_hbm.at[pl.ds(tbase + s * C, C)], osem)
                @pl.when(s + 2 < S)
                def _():
                    gather(s + 2, rows, gsem)
            return carry

        lax.fori_loop(0, S // 2, body, 0)
        for b in range(2):
            s_last = S - 2 + b
            pltpu.make_async_copy(
                bufs[b][2], out_hbm.at[pl.ds(tbase + s_last * C, C)],
                bufs[b][3]).wait()

    return sc_kernel


def kernel(value, mask, tables):
    B, T, Q = value.shape
    Qt, K, D = tables.shape
    N = B * T

    v = value.reshape(N, Q).astype(jnp.int32)
    offs = (jnp.arange(Q, dtype=jnp.int32) * K)[None, :]
    fidx = jnp.where(mask.reshape(N, 1), v + offs, Q * K).reshape(N * Q)
    tab = lax.bitcast_convert_type(
        tables.astype(jnp.bfloat16).reshape(Q * K, D // 2, 2), jnp.int32)
    tab = jnp.concatenate([tab, jnp.zeros((8, D // 2), jnp.int32)], axis=0)

    out = _build_sc_kernel(N, Q, D, 32)(tab, fidx)
    out = lax.bitcast_convert_type(out, jnp.bfloat16)
    return out.reshape(B, T, D).astype(jnp.float32)


# speedup vs baseline: 2.4737x; 2.4737x over previous
import functools

import jax
import jax.numpy as jnp
from jax import lax
from jax.experimental import pallas as pl
from jax.experimental.pallas import tpu as pltpu
from jax.experimental.pallas import tpu_sc as plsc

_LANES = 16
_NUM_CORES = 2
_NUM_SUBCORES = 16
_MAX_IDX = 128


@functools.lru_cache(maxsize=None)
def _build_sc_kernel(N, Q, K, D, C):
    NW = _NUM_CORES * _NUM_SUBCORES
    TPW = N // NW
    S = TPW // C
    R = C * Q
    G = R // _MAX_IDX

    mesh = plsc.VectorSubcoreMesh(
        core_axis_name="c", subcore_axis_name="s",
        num_cores=_NUM_CORES, num_subcores=_NUM_SUBCORES)

    @functools.partial(
        pl.kernel,
        out_type=jax.ShapeDtypeStruct((N, D), jnp.float32),
        mesh=mesh,
        scratch_types=[
            pltpu.VMEM((TPW * Q,), jnp.int32),
            pltpu.VMEM((R, D // 2), jnp.int32),
            pltpu.VMEM((R, D // 2), jnp.int32),
            pltpu.VMEM((C, D), jnp.float32),
            pltpu.VMEM((C, D), jnp.float32),
            pltpu.SemaphoreType.DMA,
            pltpu.SemaphoreType.DMA,
            pltpu.SemaphoreType.DMA,
            pltpu.SemaphoreType.DMA,
        ],
    )
    def sc_kernel(tab_hbm, val_hbm, out_hbm, fidx_v,
                  rows0, rows1, ob0, ob1, g0, g1, o0, o1):
        wid = lax.axis_index("s") * _NUM_CORES + lax.axis_index("c")
        tbase = wid * TPW
        bufs = ((rows0, g0, ob0, o0), (rows1, g1, ob1, o1))

        pltpu.sync_copy(val_hbm.at[pl.ds(tbase * Q, TPW * Q)], fidx_v)
        lanes = lax.iota(jnp.int32, _LANES)
        kbits = K.bit_length() - 1
        lq = lanes & (Q - 1)
        offp = (lq << kbits) + lq

        def build(c, carry3):
            sl = pl.ds(c * _LANES, _LANES)
            fidx_v[sl] = fidx_v[sl] + offp
            return carry3
        lax.fori_loop(0, TPW * Q // _LANES, build, 0)

        def gather(s, rows, gsem):
            for h in range(G):
                pltpu.async_copy(
                    tab_hbm.at[fidx_v.at[pl.ds(s * R + h * _MAX_IDX, _MAX_IDX)]],
                    rows.at[pl.ds(h * _MAX_IDX, _MAX_IDX)], gsem)

        def gather_wait(s, rows, gsem):
            for h in range(G):
                pltpu.make_async_copy(
                    tab_hbm.at[fidx_v.at[pl.ds(s * R + h * _MAX_IDX, _MAX_IDX)]],
                    rows.at[pl.ds(h * _MAX_IDX, _MAX_IDX)], gsem).wait()

        himask = jnp.full((_LANES,), -65536, jnp.int32)

        def compute(rows_v, ob_v):
            @plsc.parallel_loop(0, C, unroll=2)
            def token(t):
                r0 = t * Q
                for jj in range(D // (2 * _LANES)):
                    sl = pl.ds(jj * _LANES, _LANES)
                    vs = [rows_v[r0 + q, sl] for q in range(Q)]
                    los = [lax.bitcast_convert_type(v << 16, jnp.float32)
                           for v in vs]
                    his = [lax.bitcast_convert_type(v & himask, jnp.float32)
                           for v in vs]
                    while len(los) > 1:
                        los = [los[i] + los[i + 1] for i in range(0, len(los), 2)]
                        his = [his[i] + his[i + 1] for i in range(0, len(his), 2)]
                    ob_v[t, pl.ds(jj * 2 * _LANES, _LANES)] = los[0]
                    ob_v[t, pl.ds(jj * 2 * _LANES + _LANES, _LANES)] = his[0]

        gather(0, rows0, g0)
        gather(1, rows1, g1)

        def body(i, carry):
            for b in range(2):
                rows, gsem, ob, osem = bufs[b]
                s = 2 * i + b
                gather_wait(s, rows, gsem)
                @pl.when(i > 0)
                def _():
                    pltpu.make_async_copy(
                        ob, out_hbm.at[pl.ds(tbase + (s - 2) * C, C)], osem).wait()
                compute(rows, ob)
                pltpu.async_copy(ob, out_hbm.at[pl.ds(tbase + s * C, C)], osem)
                @pl.when(s + 2 < S)
                def _():
                    gather(s + 2, rows, gsem)
            return carry

        lax.fori_loop(0, S // 2, body, 0)
        for b in range(2):
            s_last = S - 2 + b
            pltpu.make_async_copy(
                bufs[b][2], out_hbm.at[pl.ds(tbase + s_last * C, C)],
                bufs[b][3]).wait()

    return sc_kernel


def kernel(value, mask, tables):
    B, T, Q = value.shape
    Qt, K, D = tables.shape
    N = B * T

    val = jnp.where(mask[..., None], value, K).reshape(N * Q).astype(jnp.int32)
    tab = tables.astype(jnp.bfloat16)
    tab = jnp.concatenate([tab, jnp.zeros((Q, 1, D), jnp.bfloat16)], axis=1)
    tab = tab.reshape(Q * (K + 1), D // 32, 2, 16).transpose(0, 1, 3, 2)
    tab = lax.bitcast_convert_type(tab, jnp.int32).reshape(Q * (K + 1), D // 2)

    out = _build_sc_kernel(N, Q, K, D, 32)(tab, val)
    return out.reshape(B, T, D)


# speedup vs baseline: 2.5257x; 1.0210x over previous
import functools

import jax
import jax.numpy as jnp
from jax import lax
from jax.experimental import pallas as pl
from jax.experimental.pallas import tpu as pltpu
from jax.experimental.pallas import tpu_sc as plsc

_LANES = 16
_NUM_CORES = 2
_NUM_SUBCORES = 16
_MAX_IDX = 128


@functools.lru_cache(maxsize=None)
def _build_sc_kernel(N, Q, K, D, C):
    NW = _NUM_CORES * _NUM_SUBCORES
    TPW = N // NW
    S = TPW // C
    R = C * Q
    G = R // _MAX_IDX

    mesh = plsc.VectorSubcoreMesh(
        core_axis_name="c", subcore_axis_name="s",
        num_cores=_NUM_CORES, num_subcores=_NUM_SUBCORES)

    @functools.partial(
        pl.kernel,
        out_type=jax.ShapeDtypeStruct((N, D), jnp.float32),
        mesh=mesh,
        scratch_types=[
            pltpu.VMEM((TPW * Q,), jnp.int32),
            pltpu.VMEM((R, D // 2), jnp.int32),
            pltpu.VMEM((R, D // 2), jnp.int32),
            pltpu.VMEM((C, D), jnp.float32),
            pltpu.VMEM((C, D), jnp.float32),
            pltpu.SemaphoreType.DMA,
            pltpu.SemaphoreType.DMA,
            pltpu.SemaphoreType.DMA,
            pltpu.SemaphoreType.DMA,
        ],
    )
    def sc_kernel(tab_hbm, val_hbm, out_hbm, fidx_v,
                  rows0, rows1, ob0, ob1, g0, g1, o0, o1):
        wid = lax.axis_index("s") * _NUM_CORES + lax.axis_index("c")
        tbase = wid * TPW
        bufs = ((rows0, g0, ob0, o0), (rows1, g1, ob1, o1))

        pltpu.sync_copy(val_hbm.at[pl.ds(tbase * Q, TPW * Q)], fidx_v)
        lanes = lax.iota(jnp.int32, _LANES)
        kbits = K.bit_length() - 1
        lq = lanes & (Q - 1)
        offp = (lq << kbits) + lq

        def build(c, carry3):
            sl = pl.ds(c * _LANES, _LANES)
            fidx_v[sl] = fidx_v[sl] + offp
            return carry3
        lax.fori_loop(0, TPW * Q // _LANES, build, 0)

        def gather(s, rows, gsem):
            for h in range(G):
                pltpu.async_copy(
                    tab_hbm.at[fidx_v.at[pl.ds(s * R + h * _MAX_IDX, _MAX_IDX)]],
                    rows.at[pl.ds(h * _MAX_IDX, _MAX_IDX)], gsem)

        def gather_wait(s, rows, gsem):
            for h in range(G):
                pltpu.make_async_copy(
                    tab_hbm.at[fidx_v.at[pl.ds(s * R + h * _MAX_IDX, _MAX_IDX)]],
                    rows.at[pl.ds(h * _MAX_IDX, _MAX_IDX)], gsem).wait()

        himask = jnp.full((_LANES,), -65536, jnp.int32)

        def compute(rows_v, ob_v):
            @plsc.parallel_loop(0, C, unroll=2)
            def token(t):
                r0 = t * Q
                for jj in range(D // (2 * _LANES)):
                    sl = pl.ds(jj * _LANES, _LANES)
                    vs = [rows_v[r0 + q, sl] for q in range(Q)]
                    los = [lax.bitcast_convert_type(v << 16, jnp.float32)
                           for v in vs]
                    his = [lax.bitcast_convert_type(v & himask, jnp.float32)
                           for v in vs]
                    while len(los) > 1:
                        los = [los[i] + los[i + 1] for i in range(0, len(los), 2)]
                        his = [his[i] + his[i + 1] for i in range(0, len(his), 2)]
                    ob_v[t, pl.ds(jj * 2 * _LANES, _LANES)] = los[0]
                    ob_v[t, pl.ds(jj * 2 * _LANES + _LANES, _LANES)] = his[0]

        gather(0, rows0, g0)
        gather(1, rows1, g1)

        def body(i, carry):
            for b in range(2):
                rows, gsem, ob, osem = bufs[b]
                s = 2 * i + b
                gather_wait(s, rows, gsem)
                @pl.when(i > 0)
                def _():
                    pltpu.make_async_copy(
                        ob, out_hbm.at[pl.ds(tbase + (s - 2) * C, C)], osem).wait()
                compute(rows, ob)
                pltpu.async_copy(ob, out_hbm.at[pl.ds(tbase + s * C, C)], osem)
                @pl.when(s + 2 < S)
                def _():
                    gather(s + 2, rows, gsem)
            return carry

        lax.fori_loop(0, S // 2, body, 0)
        for b in range(2):
            s_last = S - 2 + b
            pltpu.make_async_copy(
                bufs[b][2], out_hbm.at[pl.ds(tbase + s_last * C, C)],
                bufs[b][3]).wait()

    return sc_kernel


def kernel(value, mask, tables):
    B, T, Q = value.shape
    Qt, K, D = tables.shape
    N = B * T

    val = jnp.where(mask[..., None], value, K).reshape(N * Q).astype(jnp.int32)
    tb = jnp.concatenate([tables, jnp.zeros((Q, 1, D), jnp.float32)], axis=1)
    tb = tb.reshape(Q * (K + 1), D // 32, 2, 16)

    def bf16_bits(x):
        r = lax.bitcast_convert_type(x, jnp.int32)
        return ((r + 0x7FFF + ((r >> 16) & 1)) >> 16) & 0xFFFF

    tab = (bf16_bits(tb[:, :, 0, :])
           | (bf16_bits(tb[:, :, 1, :]) << 16)).reshape(Q * (K + 1), D // 2)

    out = _build_sc_kernel(N, Q, K, D, 32)(tab, val)
    return out.reshape(B, T, D)
